# baseline (speedup 1.0000x reference)
import functools

import jax
import jax.numpy as jnp
from jax import lax
from jax.experimental import pallas as pl
from jax.experimental.pallas import tpu as pltpu
from jax.experimental.pallas import tpu_sc as plsc

_L = 16
_NW = 32
_CH = 64


def _lane_splat(vec, lane):
    idx = jnp.full((_L,), lane, dtype=jnp.int32)
    return vec.at[idx].get(mode="promise_in_bounds")


def _sum_splat(vec):
    total = _lane_splat(vec, 0)
    for l in range(1, _L):
        total = total + _lane_splat(vec, l)
    return total


def kernel(x, Cost, Pd, w_capacity, Pmax):
    B = x.shape[0]
    NU = Cost.shape[0]
    NL = Pd.shape[0]
    RB = B // _NW
    w16 = jnp.broadcast_to(w_capacity.astype(jnp.float32), (_L,))

    mesh = plsc.VectorSubcoreMesh(core_axis_name="c", subcore_axis_name="s")

    @functools.partial(
        pl.kernel,
        mesh=mesh,
        out_type=jax.ShapeDtypeStruct((B, NU), jnp.float32),
        scratch_types=[
            pltpu.VMEM((NU,), jnp.float32),
            pltpu.VMEM((NU,), jnp.float32),
            pltpu.VMEM((NU,), jnp.float32),
            pltpu.VMEM((NL,), jnp.float32),
            pltpu.VMEM((_L,), jnp.float32),
            pltpu.VMEM((RB,), jnp.float32),
            pltpu.VMEM((RB,), jnp.float32),
            pltpu.VMEM((_CH, NU), jnp.float32),
            pltpu.VMEM_SHARED((NU,), jnp.float32),
        ],
    )
    def run(x_hbm, cost_hbm, pd_hbm, w_hbm, pmax_hbm, out_hbm,
            cost_v, pmax_v, cb_v, pd_v, w_v, x_v, td_v, outbuf, cb_sh):
        c = lax.axis_index("c")
        s = lax.axis_index("s")
        wid = s * 2 + c
        base = pl.multiple_of(wid * RB, RB)

        pltpu.sync_copy(cost_hbm, cost_v)
        pltpu.sync_copy(pmax_hbm, pmax_v)
        pltpu.sync_copy(pd_hbm, pd_v)
        pltpu.sync_copy(w_hbm, w_v)
        pltpu.sync_copy(x_hbm.at[pl.ds(base, RB)], x_v)

        iota = lax.broadcasted_iota(jnp.int32, (_L,), 0)

        w_vec = w_v[...]
        acc = jnp.zeros((_L,), jnp.float32)
        for k in range(NL // _L):
            acc = acc + pd_v[pl.ds(k * _L, _L)]
        spd_vec = _sum_splat(acc)

        for g in range(RB // _L):
            x_vec = x_v[pl.ds(g * _L, _L)]
            td_v[pl.ds(g * _L, _L)] = spd_vec - w_vec * x_vec

        gb = pl.multiple_of(s * (NU // _L * _L // 16), 32)
        for gv in range(2):
            g0 = gb + gv * _L
            cost_g = cost_v[pl.ds(g0, _L)]
            g_ids = g0 + iota

            def jbody(jg, acc_cb):
                cj = cost_v[pl.ds(jg * _L, _L)]
                pj = pmax_v[pl.ds(jg * _L, _L)]
                jb = jg * _L
                for l in range(_L):
                    cjb = _lane_splat(cj, l)
                    pjb = _lane_splat(pj, l)
                    before = (cjb < cost_g) | (
                        (cjb == cost_g) & (jb + l < g_ids))
                    acc_cb = acc_cb + jnp.where(before, pjb, 0.0)
                return acc_cb

            cb_chunk = lax.fori_loop(0, NU // _L, jbody,
                                     jnp.zeros((_L,), jnp.float32))
            cb_v[pl.ds(g0, _L)] = cb_chunk

        pltpu.sync_copy(cb_v.at[pl.ds(gb, 32)], cb_sh.at[pl.ds(gb, 32)])
        plsc.subcore_barrier()
        pltpu.sync_copy(cb_sh, cb_v)

        def chunk_body(ch, carry):
            row0 = ch * _CH
            for half in range(2):
                cbs = [cb_v[pl.ds(half * 256 + v * _L, _L)]
                       for v in range(16)]
                pms = [pmax_v[pl.ds(half * 256 + v * _L, _L)]
                       for v in range(16)]

                def grp_body(gi, carry2):
                    td_vec = td_v[pl.ds(row0 + gi * _L, _L)]
                    for l in range(_L):
                        tdb = _lane_splat(td_vec, l)
                        r = gi * _L + l
                        for v in range(16):
                            col = half * 256 + v * _L
                            outbuf[r, pl.ds(col, _L)] = jnp.minimum(
                                jnp.maximum(tdb - cbs[v], 0.0), pms[v])
                    return carry2

                lax.fori_loop(0, _CH // _L, grp_body, 0)
            pltpu.sync_copy(outbuf, out_hbm.at[pl.ds(base + row0, _CH)])
            return carry

        lax.fori_loop(0, RB // _CH, chunk_body, 0)

    return run(x, Cost, Pd, w16, Pmax)


# speedup vs baseline: 1.1907x; 1.1907x over previous
import functools

import jax
import jax.numpy as jnp
from jax import lax
from jax.experimental import pallas as pl
from jax.experimental.pallas import tpu as pltpu
from jax.experimental.pallas import tpu_sc as plsc

_L = 16
_NW = 32
_CH = 64


def _lane_splat(vec, lane):
    idx = jnp.full((_L,), lane, dtype=jnp.int32)
    return vec.at[idx].get(mode="promise_in_bounds")


def _sum_splat(vec):
    total = _lane_splat(vec, 0)
    for l in range(1, _L):
        total = total + _lane_splat(vec, l)
    return total


def kernel(x, Cost, Pd, w_capacity, Pmax):
    B = x.shape[0]
    NU = Cost.shape[0]
    NL = Pd.shape[0]
    RB = B // _NW
    w16 = jnp.broadcast_to(w_capacity.astype(jnp.float32), (_L,))

    mesh = plsc.VectorSubcoreMesh(core_axis_name="c", subcore_axis_name="s")

    @functools.partial(
        pl.kernel,
        mesh=mesh,
        out_type=jax.ShapeDtypeStruct((B, NU), jnp.float32),
        scratch_types=[
            pltpu.VMEM((NU,), jnp.float32),
            pltpu.VMEM((NU,), jnp.float32),
            pltpu.VMEM((NU,), jnp.float32),
            pltpu.VMEM((NL,), jnp.float32),
            pltpu.VMEM((_L,), jnp.float32),
            pltpu.VMEM((RB,), jnp.float32),
            pltpu.VMEM((RB,), jnp.float32),
            pltpu.VMEM((2, _CH, NU), jnp.float32),
            pltpu.VMEM_SHARED((NU,), jnp.float32),
            pltpu.SemaphoreType.DMA,
        ],
    )
    def run(x_hbm, cost_hbm, pd_hbm, w_hbm, pmax_hbm, out_hbm,
            cost_v, pmax_v, cb_v, pd_v, w_v, x_v, td_v, outbuf, cb_sh, sem):
        c = lax.axis_index("c")
        s = lax.axis_index("s")
        wid = s * 2 + c
        base = pl.multiple_of(wid * RB, RB)

        pltpu.sync_copy(cost_hbm, cost_v)
        pltpu.sync_copy(pmax_hbm, pmax_v)
        pltpu.sync_copy(pd_hbm, pd_v)
        pltpu.sync_copy(w_hbm, w_v)
        pltpu.sync_copy(x_hbm.at[pl.ds(base, RB)], x_v)

        iota = lax.broadcasted_iota(jnp.int32, (_L,), 0)

        w_vec = w_v[...]
        acc = jnp.zeros((_L,), jnp.float32)
        for k in range(NL // _L):
            acc = acc + pd_v[pl.ds(k * _L, _L)]
        spd_vec = _sum_splat(acc)

        for g in range(RB // _L):
            x_vec = x_v[pl.ds(g * _L, _L)]
            td_v[pl.ds(g * _L, _L)] = spd_vec - w_vec * x_vec

        gb = pl.multiple_of(s * (NU // _L * _L // 16), 32)
        for gv in range(2):
            g0 = gb + gv * _L
            cost_g = cost_v[pl.ds(g0, _L)]
            g_ids = g0 + iota

            def jbody(jg, acc_cb):
                cj = cost_v[pl.ds(jg * _L, _L)]
                pj = pmax_v[pl.ds(jg * _L, _L)]
                jb = jg * _L
                for l in range(_L):
                    cjb = _lane_splat(cj, l)
                    pjb = _lane_splat(pj, l)
                    before = (cjb < cost_g) | (
                        (cjb == cost_g) & (jb + l < g_ids))
                    acc_cb = acc_cb + jnp.where(before, pjb, 0.0)
                return acc_cb

            cb_chunk = lax.fori_loop(0, NU // _L, jbody,
                                     jnp.zeros((_L,), jnp.float32))
            cb_v[pl.ds(g0, _L)] = cb_chunk

        pltpu.sync_copy(cb_v.at[pl.ds(gb, 32)], cb_sh.at[pl.ds(gb, 32)])
        plsc.subcore_barrier()
        pltpu.sync_copy(cb_sh, cb_v)

        n_chunks = RB // _CH

        def out_copy(ch):
            parity = ch & 1
            return pltpu.make_async_copy(
                outbuf.at[parity],
                out_hbm.at[pl.ds(base + ch * _CH, _CH)],
                sem)

        def chunk_body(ch, carry):
            row0 = ch * _CH
            parity = ch & 1

            @pl.when(ch >= 2)
            def _():
                out_copy(ch - 2).wait()

            for half in range(2):
                cbs = [cb_v[pl.ds(half * 256 + v * _L, _L)]
                       for v in range(16)]
                pms = [pmax_v[pl.ds(half * 256 + v * _L, _L)]
                       for v in range(16)]

                def grp_body(gi, carry2):
                    td_vec = td_v[pl.ds(row0 + gi * _L, _L)]
                    for l in range(_L):
                        tdb = _lane_splat(td_vec, l)
                        r = gi * _L + l
                        for v in range(16):
                            col = half * 256 + v * _L
                            outbuf[parity, r, pl.ds(col, _L)] = jnp.minimum(
                                jnp.maximum(tdb - cbs[v], 0.0), pms[v])
                    return carry2

                lax.fori_loop(0, _CH // _L, grp_body, 0)
            out_copy(ch).start()
            return carry

        lax.fori_loop(0, n_chunks, chunk_body, 0)
        out_copy(n_chunks - 2).wait()
        out_copy(n_chunks - 1).wait()

    return run(x, Cost, Pd, w16, Pmax)


# speedup vs baseline: 2.0020x; 1.6814x over previous
import jax
import jax.numpy as jnp
from jax import lax
from jax.experimental import pallas as pl
from jax.experimental.pallas import tpu as pltpu

_RB = 4096


def _body(x_ref, pd_ref, cost_col_ref, cost_row_ref, pmax_col_ref,
          pmax_row_ref, w_ref, out_ref, cb_ref):
    n_unit = cost_col_ref.shape[0]

    @pl.when(pl.program_id(0) == 0)
    def _():
        cc = cost_col_ref[...]
        cr = cost_row_ref[...]
        jcol = lax.broadcasted_iota(jnp.int32, (n_unit, n_unit), 0)
        grow = lax.broadcasted_iota(jnp.int32, (n_unit, n_unit), 1)
        before = (cc < cr) | ((cc == cr) & (jcol < grow))
        cb_ref[...] = jnp.sum(
            jnp.where(before, pmax_col_ref[...], 0.0), axis=0, keepdims=True)

    total_d = jnp.sum(pd_ref[...]) - w_ref[0, 0] * x_ref[...]
    out_ref[...] = jnp.clip(total_d - cb_ref[...], 0.0, pmax_row_ref[...])


def kernel(x, Cost, Pd, w_capacity, Pmax):
    B = x.shape[0]
    n_unit = Cost.shape[0]
    x_col = x.reshape(B, 1)
    pd2d = Pd.reshape(-1, 128)
    cost_col = Cost.reshape(n_unit, 1)
    cost_row = Cost.reshape(1, n_unit)
    pmax_col = Pmax.reshape(n_unit, 1)
    pmax_row = Pmax.reshape(1, n_unit)
    w2d = w_capacity.reshape(1, 1)

    grid = (B // _RB,)
    return pl.pallas_call(
        _body,
        grid=grid,
        in_specs=[
            pl.BlockSpec((_RB, 1), lambda i: (i, 0)),
            pl.BlockSpec(pd2d.shape, lambda i: (0, 0)),
            pl.BlockSpec((n_unit, 1), lambda i: (0, 0)),
            pl.BlockSpec((1, n_unit), lambda i: (0, 0)),
            pl.BlockSpec((n_unit, 1), lambda i: (0, 0)),
            pl.BlockSpec((1, n_unit), lambda i: (0, 0)),
            pl.BlockSpec((1, 1), lambda i: (0, 0)),
        ],
        out_specs=pl.BlockSpec((_RB, n_unit), lambda i: (i, 0)),
        out_shape=jax.ShapeDtypeStruct((B, n_unit), jnp.float32),
        scratch_shapes=[pltpu.VMEM((1, n_unit), jnp.float32)],
        compiler_params=pltpu.CompilerParams(
            dimension_semantics=("arbitrary",)),
    )(x_col, pd2d, cost_col, cost_row, pmax_col, pmax_row, w2d)
